# baseline (speedup 1.0000x reference)
import functools

import jax
import jax.numpy as jnp
from jax import lax
from jax.experimental import pallas as pl
from jax.experimental.pallas import tpu as pltpu
from jax.experimental.pallas import tpu_sc as plsc

NC = 2
NS = 16
LANES = 16
TILE = 16


def _sc_body(B, S, D, xf, idb, pe_hbm, tp_hbm, outf,
             x_b, o_b, pe_all, tp_v, ids_b, sem_x, sem_i, sem_o, sem_pe):
    nw = NC * NS
    seq_per_w = S // nw
    n_st = seq_per_w // TILE
    n_rounds = B * n_st
    groups = D // LANES
    wid = lax.axis_index("s") * NC + lax.axis_index("c")
    seq0 = wid * seq_per_w

    h_pe = pltpu.async_copy(pe_hbm.at[pl.ds(seq0, seq_per_w)], pe_all, sem_pe)
    pltpu.sync_copy(tp_hbm, tp_v)
    h_pe.wait()

    def row_of(r):
        b = r // n_st
        st = r % n_st
        return b * S + seq0 + st * TILE, st

    def issue_in(r, p):
        row0, _ = row_of(r)
        pltpu.async_copy(xf.at[pl.ds(row0, TILE)], x_b[p], sem_x[p])
        pltpu.async_copy(idb.at[pl.ds(row0, TILE)], ids_b[p], sem_i[p])

    issue_in(0, 0)
    issue_in(1, 1)

    def pair(k, carry):
        for j in range(2):
            r = 2 * k + j
            row0, st = row_of(r)
            xr, orr = x_b[j], o_b[j]
            idvs = [ids_b[j][t, :] for t in range(TILE)]
            pe_row0 = st * TILE

            @plsc.parallel_loop(0, groups, unroll=2)
            def grp(g):
                sl = pl.ds(g * LANES, LANES)
                t0, t1, t2, t3 = (tp_v[kk, sl] for kk in range(4))
                for t in range(TILE):
                    idv = idvs[t]
                    lo = jnp.where(idv == 0, t0, t1)
                    hi = jnp.where(idv == 2, t2, t3)
                    tp_row = jnp.where(idv < 2, lo, hi)
                    orr[t, sl] = (xr[t, sl] + pe_all[pe_row0 + t, sl]
                                  + tp_row)

        return carry

    lax.fori_loop(0, n_rounds // 2, pair, 0)
    for j in range(2):
        pltpu.make_async_copy(xf.at[pl.ds(0, TILE)], x_b[j], sem_x[j]).wait()
        pltpu.make_async_copy(idb.at[pl.ds(0, TILE)], ids_b[j],
                              sem_i[j]).wait()
        pltpu.async_copy(o_b[j], outf.at[pl.ds(j * TILE, TILE)],
                         sem_o[j]).wait()


def kernel(x, type_ids, pe, type_phase):
    B, S, D = x.shape
    xf = x.reshape(B * S, D)
    idb = jnp.broadcast_to(type_ids.reshape(B * S, 1).astype(jnp.int32),
                           (B * S, LANES))
    pe_s = pe[:S]
    seq_per_w = S // (NC * NS)

    mesh = plsc.VectorSubcoreMesh(core_axis_name="c", subcore_axis_name="s",
                                  num_cores=NC, num_subcores=NS)
    run = pl.kernel(
        functools.partial(_sc_body, B, S, D),
        out_type=jax.ShapeDtypeStruct((B * S, D), jnp.float32),
        mesh=mesh,
        scratch_types=[
            [pltpu.VMEM((TILE, D), jnp.float32) for _ in range(2)],
            [pltpu.VMEM((TILE, D), jnp.float32) for _ in range(2)],
            pltpu.VMEM((seq_per_w, D), jnp.float32),
            pltpu.VMEM(type_phase.shape, jnp.float32),
            [pltpu.VMEM((TILE, LANES), jnp.int32) for _ in range(2)],
            [pltpu.SemaphoreType.DMA for _ in range(2)],
            [pltpu.SemaphoreType.DMA for _ in range(2)],
            [pltpu.SemaphoreType.DMA for _ in range(2)],
            pltpu.SemaphoreType.DMA,
        ],
    )
    out = run(xf, idb, pe_s, type_phase)
    return out.reshape(B, S, D)


# speedup vs baseline: 1.5052x; 1.5052x over previous
import functools

import jax
import jax.numpy as jnp
from jax import lax
from jax.experimental import pallas as pl
from jax.experimental.pallas import tpu as pltpu
from jax.experimental.pallas import tpu_sc as plsc

NC = 2
NS = 16
LANES = 16
TILE = 16


def _sc_body(B, S, D, xf, idsf, pe_hbm, tp_hbm, outf,
             x_b, o_b, pe_all, tp_v, ids_sh, ids_sm, sem_x, sem_o, sem_pe):
    nw = NC * NS
    seq_per_w = S // nw
    n_st = seq_per_w // TILE
    n_rounds = B * n_st
    groups = D // LANES
    sid = lax.axis_index("s")
    wid = sid * NC + lax.axis_index("c")
    seq0 = wid * seq_per_w

    h_pe = pltpu.async_copy(pe_hbm.at[pl.ds(seq0, seq_per_w)], pe_all, sem_pe)
    pltpu.sync_copy(tp_hbm, tp_v)

    @pl.when(sid == 0)
    def _():
        pltpu.sync_copy(idsf, ids_sh)
    plsc.subcore_barrier()
    for b in range(B):
        pltpu.sync_copy(ids_sh.at[pl.ds(b * S + seq0, seq_per_w)],
                        ids_sm.at[pl.ds(b * seq_per_w, seq_per_w)])
    h_pe.wait()

    def row_of(r):
        b = r // n_st
        st = r % n_st
        return b * S + seq0 + st * TILE, b * seq_per_w + st * TILE, st

    def issue_in(r, p):
        row0, _, _ = row_of(r)
        pltpu.async_copy(xf.at[pl.ds(row0, TILE)], x_b[p], sem_x[p])

    issue_in(0, 0)
    issue_in(1, 1)

    def pair(k, carry):
        for j in range(2):
            r = 2 * k + j
            row0, loc0, st = row_of(r)
            pltpu.make_async_copy(xf.at[pl.ds(0, TILE)], x_b[j],
                                  sem_x[j]).wait()

            @pl.when(r >= 2)
            def _():
                pltpu.make_async_copy(o_b[j], outf.at[pl.ds(0, TILE)],
                                      sem_o[j]).wait()

            xr, orr = x_b[j], o_b[j]
            tids = [ids_sm[loc0 + t] for t in range(TILE)]
            pe_row0 = st * TILE

            @plsc.parallel_loop(0, groups, unroll=2)
            def grp(g):
                sl = pl.ds(g * LANES, LANES)
                for t in range(TILE):
                    orr[t, sl] = (xr[t, sl] + pe_all[pe_row0 + t, sl]
                                  + tp_v[tids[t], sl])

            pltpu.async_copy(orr, outf.at[pl.ds(row0, TILE)], sem_o[j])

            @pl.when(r + 2 < n_rounds)
            def _():
                r2row0, _, _ = row_of(r + 2)
                pltpu.async_copy(xf.at[pl.ds(r2row0, TILE)], x_b[j], sem_x[j])
        return carry

    lax.fori_loop(0, n_rounds // 2, pair, 0)
    for j in range(2):
        pltpu.make_async_copy(o_b[j], outf.at[pl.ds(0, TILE)], sem_o[j]).wait()


def kernel(x, type_ids, pe, type_phase):
    B, S, D = x.shape
    xf = x.reshape(B * S, D)
    idsf = type_ids.reshape(B * S).astype(jnp.int32)
    pe_s = pe[:S]
    seq_per_w = S // (NC * NS)

    mesh = plsc.VectorSubcoreMesh(core_axis_name="c", subcore_axis_name="s",
                                  num_cores=NC, num_subcores=NS)
    run = pl.kernel(
        functools.partial(_sc_body, B, S, D),
        out_type=jax.ShapeDtypeStruct((B * S, D), jnp.float32),
        mesh=mesh,
        scratch_types=[
            [pltpu.VMEM((TILE, D), jnp.float32) for _ in range(2)],
            [pltpu.VMEM((TILE, D), jnp.float32) for _ in range(2)],
            pltpu.VMEM((seq_per_w, D), jnp.float32),
            pltpu.VMEM(type_phase.shape, jnp.float32),
            pltpu.VMEM_SHARED((B * S,), jnp.int32),
            pltpu.SMEM((B * seq_per_w,), jnp.int32),
            [pltpu.SemaphoreType.DMA for _ in range(2)],
            [pltpu.SemaphoreType.DMA for _ in range(2)],
            pltpu.SemaphoreType.DMA,
        ],
    )
    out = run(xf, idsf, pe_s, type_phase)
    return out.reshape(B, S, D)
